# Initial kernel scaffold; baseline (speedup 1.0000x reference)
import jax
import jax.numpy as jnp
from jax.experimental import pallas as pl


def kernel(node_feature, type_feature, length_feature, lane_feature, edge_index, struct_adj, struct_assign, fnc_assign, params):
    raise NotImplementedError("write your pallas kernel here")


# speedup vs baseline: 5.2139x; 5.2139x over previous
import functools

import jax
import jax.numpy as jnp
from jax import lax
from jax.experimental import pallas as pl
from jax.experimental.pallas import tpu as pltpu
from jax.experimental.pallas import tpu_sc as plsc

N = 10000
NPAD = 10240
E = 320000
D = 128
CH = 128
NCHUNK = E // CH
NC = 2
NT = 16
ROWS_PER_TILE = NPAD // NT
HB = NPAD // D

_mesh = plsc.VectorSubcoreMesh(core_axis_name="c", subcore_axis_name="s")

_f32 = jnp.float32
_i32 = jnp.int32


def _embed_body(nf, tf, lf, af, nt, tt, lt, at_, out,
                idx_v, b0, b1, b2, b3, sem):
    wid = lax.axis_index("s") * NC + lax.axis_index("c")
    for k in range(5):
        base = wid * 320 + k * 64
        rsl = pl.ds(base, 64)
        pltpu.sync_copy(nf.at[rsl], idx_v)
        pltpu.async_copy(nt.at[idx_v], b0, sem).wait()
        pltpu.sync_copy(b0, out.at[0, rsl])
        pltpu.sync_copy(tf.at[rsl], idx_v)
        pltpu.async_copy(tt.at[idx_v], b1, sem).wait()
        pltpu.sync_copy(b1, out.at[1, rsl])
        pltpu.sync_copy(lf.at[rsl], idx_v)
        pltpu.async_copy(lt.at[idx_v], b2, sem).wait()
        pltpu.sync_copy(b2, out.at[2, rsl])
        pltpu.sync_copy(af.at[rsl], idx_v)
        pltpu.async_copy(at_.at[idx_v], b3, sem).wait()
        pltpu.sync_copy(b3, out.at[3, rsl])


def _embed(nf, tf, lf, af, nt, tt, lt, at_):
    return pl.kernel(
        _embed_body,
        out_type=jax.ShapeDtypeStruct((4, NPAD, D), _f32),
        mesh=_mesh,
        scratch_types=[
            pltpu.VMEM((64,), _i32),
            pltpu.VMEM((64, D), _f32),
            pltpu.VMEM((64, D), _f32),
            pltpu.VMEM((64, D), _f32),
            pltpu.VMEM((64, D), _f32),
            pltpu.SemaphoreType.DMA,
        ],
    )(nf, tf, lf, af, nt, tt, lt, at_)


def _t0_body(a_ref, o_ref):
    o_ref[...] = jnp.concatenate(
        [a_ref[0, :, :64], a_ref[1, :, :32],
         a_ref[2, :, :16], a_ref[3, :, :16]], axis=1)


def _t0(g):
    blk = 640
    return pl.pallas_call(
        _t0_body,
        grid=(NPAD // blk,),
        in_specs=[pl.BlockSpec((4, blk, D), lambda i: (0, i, 0))],
        out_specs=pl.BlockSpec((blk, D), lambda i: (i, 0)),
        out_shape=jax.ShapeDtypeStruct((NPAD, D), _f32),
    )(g)


def _segsum_body(with_hist, x, rows2, cols2, zeros, zrow, *args):
    if with_hist:
        out, out_h0, out_h1, acc, acc_h, rows_v, cols_v, vals, ones_v, sem = args
    else:
        out, acc, rows_v, cols_v, vals, sem = args
    c = lax.axis_index("c")
    s = lax.axis_index("s")
    rsl = pl.ds(ROWS_PER_TILE * s, ROWS_PER_TILE)
    pltpu.sync_copy(zeros.at[rsl], acc.at[rsl])
    if with_hist:
        for k in range(CH // 16):
            ones_v[pl.ds(16 * k, 16)] = jnp.full((16,), 1.0, _f32)

        @pl.when(s == 0)
        def _():
            pltpu.sync_copy(zrow, acc_h)

    plsc.subcore_barrier()
    base = 1250 * c + 78 * s + jnp.minimum(s, 2)
    cnt = jnp.where(s < 2, 79, 78)

    def step(j, carry):
        ch = base + j
        pltpu.sync_copy(rows2.at[ch], rows_v)
        pltpu.sync_copy(cols2.at[ch], cols_v)
        pltpu.async_copy(x.at[cols_v], vals, sem).wait()
        pltpu.sync_copy(vals, acc.at[rows_v], add=True)
        if with_hist:
            pltpu.sync_copy(ones_v, acc_h.at[rows_v], add=True)
        return carry

    lax.fori_loop(0, cnt, step, 0)
    plsc.subcore_barrier()
    pltpu.sync_copy(acc.at[rsl], out.at[c, rsl])
    if with_hist:
        @pl.when(c == 0)
        def _():
            pltpu.sync_copy(acc_h.at[rsl], out_h0.at[rsl])

        @pl.when(c == 1)
        def _():
            pltpu.sync_copy(acc_h.at[rsl], out_h1.at[rsl])


def _segsum(x, rows2, cols2, zeros, zrow, with_hist):
    out_type = [jax.ShapeDtypeStruct((2, NPAD, D), _f32)]
    scratch = [
        pltpu.VMEM_SHARED((NPAD, D), _f32),
        pltpu.VMEM((CH,), _i32),
        pltpu.VMEM((CH,), _i32),
        pltpu.VMEM((CH, D), _f32),
        pltpu.SemaphoreType.DMA,
    ]
    if with_hist:
        out_type += [jax.ShapeDtypeStruct((NPAD,), _f32),
                     jax.ShapeDtypeStruct((NPAD,), _f32)]
        scratch = [
            scratch[0],
            pltpu.VMEM_SHARED((NPAD,), _f32),
            scratch[1], scratch[2], scratch[3],
            pltpu.VMEM((CH,), _f32),
            pltpu.SemaphoreType.DMA,
        ]
    res = pl.kernel(
        functools.partial(_segsum_body, with_hist),
        out_type=out_type,
        mesh=_mesh,
        scratch_types=scratch,
    )(x, rows2, cols2, zeros, zrow)
    return res if with_hist else res[0]


def _t2_body(a_ref, rs_ref, w_ref, o_ref):
    s = a_ref[0] + a_ref[1]
    z = jnp.dot(s, w_ref[...], preferred_element_type=_f32) / rs_ref[...]
    o_ref[...] = jnp.where(z > 0, z, jnp.exp(z) - 1.0)


def _t2(acc, rs, w, out_rows, blk):
    return pl.pallas_call(
        _t2_body,
        grid=(out_rows // blk,),
        in_specs=[
            pl.BlockSpec((2, blk, D), lambda i: (0, i, 0)),
            pl.BlockSpec((blk, 1), lambda i: (i, 0)),
            pl.BlockSpec((D, D), lambda i: (0, 0)),
        ],
        out_specs=pl.BlockSpec((blk, D), lambda i: (i, 0)),
        out_shape=jax.ShapeDtypeStruct((out_rows, D), _f32),
    )(acc, rs, w)


def kernel(node_feature, type_feature, length_feature, lane_feature,
           edge_index, struct_adj, struct_assign, fnc_assign, params):
    del struct_adj, struct_assign, fnc_assign

    def pad_idx(a):
        return jnp.pad(a.astype(_i32), (0, NPAD - N))

    def pad_tab(t):
        t = t.astype(_f32)
        return jnp.pad(t, ((0, 0), (0, D - t.shape[1])))

    nf = pad_idx(node_feature)
    tf = pad_idx(type_feature)
    lf = pad_idx(length_feature)
    af = pad_idx(lane_feature)
    rows2 = edge_index[0].astype(_i32).reshape(NCHUNK, CH)
    cols2 = edge_index[1].astype(_i32).reshape(NCHUNK, CH)

    z128 = jnp.zeros((NPAD, D), _f32)
    zrow = jnp.zeros((NPAD,), _f32)

    g = _embed(nf, tf, lf, af,
               pad_tab(params["node_table"]),
               pad_tab(params["type_table"]),
               pad_tab(params["length_table"]),
               pad_tab(params["lane_table"]))
    x1 = _t0(g)

    ws = [p["gat_W"].astype(_f32) for p in params["layers"]]

    acc1, h0, h1 = _segsum(x1, rows2, cols2, z128, zrow, True)
    rs = (h0 + h1).reshape(NPAD, 1)
    x2 = _t2(acc1, rs, ws[0], NPAD, 640)
    acc2 = _segsum(x2, rows2, cols2, z128, zrow, False)
    x3 = _t2(acc2, rs, ws[1], NPAD, 640)
    acc3 = _segsum(x3, rows2, cols2, z128, zrow, False)
    return _t2(acc3, rs, ws[2], N, 400)


# speedup vs baseline: 6.6126x; 1.2683x over previous
import functools

import jax
import jax.numpy as jnp
from jax import lax
from jax.experimental import pallas as pl
from jax.experimental.pallas import tpu as pltpu
from jax.experimental.pallas import tpu_sc as plsc

N = 10000
NPAD = 10240
E = 320000
D = 128
CH = 64
NCHUNK = 5120
EPAD = NCHUNK * CH - E
NC = 2
NT = 16
CPT = NCHUNK // (NC * NT)
PH = CPT // 2
ACCR = 10112
RPT = ACCR // NT

_mesh = plsc.VectorSubcoreMesh(core_axis_name="c", subcore_axis_name="s")

_f32 = jnp.float32
_i32 = jnp.int32


def _embed_body(nf, tf, lf, af, nt, tt, lt, at_, out,
                idx_v, b0, b1, b2, b3, sem):
    wid = lax.axis_index("s") * NC + lax.axis_index("c")
    for k in range(5):
        base = wid * 320 + k * 64
        rsl = pl.ds(base, 64)
        pltpu.sync_copy(nf.at[rsl], idx_v)
        pltpu.async_copy(nt.at[idx_v], b0, sem).wait()
        pltpu.sync_copy(b0, out.at[0, rsl])
        pltpu.sync_copy(tf.at[rsl], idx_v)
        pltpu.async_copy(tt.at[idx_v], b1, sem).wait()
        pltpu.sync_copy(b1, out.at[1, rsl])
        pltpu.sync_copy(lf.at[rsl], idx_v)
        pltpu.async_copy(lt.at[idx_v], b2, sem).wait()
        pltpu.sync_copy(b2, out.at[2, rsl])
        pltpu.sync_copy(af.at[rsl], idx_v)
        pltpu.async_copy(at_.at[idx_v], b3, sem).wait()
        pltpu.sync_copy(b3, out.at[3, rsl])


def _embed(nf, tf, lf, af, nt, tt, lt, at_):
    return pl.kernel(
        _embed_body,
        out_type=jax.ShapeDtypeStruct((4, NPAD, D), _f32),
        mesh=_mesh,
        scratch_types=[
            pltpu.VMEM((64,), _i32),
            pltpu.VMEM((64, D), _f32),
            pltpu.VMEM((64, D), _f32),
            pltpu.VMEM((64, D), _f32),
            pltpu.VMEM((64, D), _f32),
            pltpu.SemaphoreType.DMA,
        ],
    )(nf, tf, lf, af, nt, tt, lt, at_)


def _t0_body(a_ref, o_ref):
    o_ref[...] = jnp.concatenate(
        [a_ref[0, :, :64], a_ref[1, :, :32],
         a_ref[2, :, :16], a_ref[3, :, :16]], axis=1)


def _t0(g):
    blk = 640
    return pl.pallas_call(
        _t0_body,
        grid=(NPAD // blk,),
        in_specs=[pl.BlockSpec((4, blk, D), lambda i: (0, i, 0))],
        out_specs=pl.BlockSpec((blk, D), lambda i: (i, 0)),
        out_shape=jax.ShapeDtypeStruct((NPAD, D), _f32),
    )(g)


def _segsum_body(with_hist, x, rows2, cols2, zeros, zrow, *args):
    if with_hist:
        (out, out_h0, out_h1, acc, acc_h, rows_all, cols_all,
         vals0, vals1, ones_v, sg0, sg1, ss0, ss1, sh0, sh1) = args
    else:
        (out, acc, rows_all, cols_all,
         vals0, vals1, sg0, sg1, ss0, ss1) = args
    vals = (vals0, vals1)
    sg = (sg0, sg1)
    ss = (ss0, ss1)
    if with_hist:
        sh = (sh0, sh1)
    c = lax.axis_index("c")
    s = lax.axis_index("s")
    rsl = pl.ds(RPT * s, RPT)
    base = (NT * c + s) * CPT
    pltpu.sync_copy(zeros.at[rsl], acc.at[rsl])
    if with_hist:
        for k in range(CH // 16):
            ones_v[pl.ds(16 * k, 16)] = jnp.full((16,), 1.0, _f32)

        @pl.when(s == 0)
        def _():
            pltpu.sync_copy(zrow.at[pl.ds(0, ACCR)], acc_h)

    plsc.subcore_barrier()

    def gather_start(j, b):
        pltpu.async_copy(x.at[cols_all.at[j]], vals[b], sg[b])

    def gather_wait(j, b):
        pltpu.make_async_copy(x.at[cols_all.at[j]], vals[b], sg[b]).wait()

    def scatter_start(j, b):
        pltpu.async_copy(vals[b], acc.at[rows_all.at[j]], ss[b], add=True)
        if with_hist:
            pltpu.async_copy(ones_v, acc_h.at[rows_all.at[j]], sh[b],
                             add=True)

    def scatter_wait(j, b):
        pltpu.make_async_copy(vals[b], acc.at[rows_all.at[j]], ss[b]).wait()
        if with_hist:
            pltpu.make_async_copy(ones_v, acc_h.at[rows_all.at[j]],
                                  sh[b]).wait()

    def step(ji, carry):
        for u in (0, 1):
            j = 2 * ji + u
            nb = 1 - u
            gather_wait(j, u)
            if u == 0:
                @pl.when(j >= 1)
                def _():
                    scatter_wait(j - 1, nb)

                gather_start(j + 1, nb)
            else:
                scatter_wait(j - 1, nb)

                @pl.when(j + 1 < PH)
                def _():
                    gather_start(j + 1, nb)

            scatter_start(j, u)
        return carry

    for p in range(CPT // PH):
        psl = pl.ds(base + p * PH, PH)
        pltpu.sync_copy(rows2.at[psl], rows_all)
        pltpu.sync_copy(cols2.at[psl], cols_all)
        gather_start(0, 0)
        lax.fori_loop(0, PH // 2, step, 0)
        scatter_wait(PH - 1, 1)
    plsc.subcore_barrier()
    pltpu.sync_copy(acc.at[rsl], out.at[c, rsl])
    if with_hist:
        for oh, cc in ((out_h0, 0), (out_h1, 1)):
            @pl.when((c == cc) & (s < 15))
            def _(oh=oh):
                hsl = pl.ds(640 * s, 640)
                pltpu.sync_copy(acc_h.at[hsl], oh.at[hsl])

            @pl.when((c == cc) & (s == 15))
            def _(oh=oh):
                hsl = pl.ds(9600, 512)
                pltpu.sync_copy(acc_h.at[hsl], oh.at[hsl])


def _segsum(x, rows2, cols2, zeros, zrow, with_hist):
    out_type = [jax.ShapeDtypeStruct((2, NPAD, D), _f32)]
    scratch = [
        pltpu.VMEM_SHARED((ACCR, D), _f32),
        pltpu.VMEM((PH, CH), _i32),
        pltpu.VMEM((PH, CH), _i32),
        pltpu.VMEM((CH, D), _f32),
        pltpu.VMEM((CH, D), _f32),
        pltpu.SemaphoreType.DMA,
        pltpu.SemaphoreType.DMA,
        pltpu.SemaphoreType.DMA,
        pltpu.SemaphoreType.DMA,
    ]
    if with_hist:
        out_type += [jax.ShapeDtypeStruct((NPAD,), _f32),
                     jax.ShapeDtypeStruct((NPAD,), _f32)]
        scratch = ([scratch[0], pltpu.VMEM_SHARED((ACCR,), _f32)]
                   + scratch[1:5] + [pltpu.VMEM((CH,), _f32)]
                   + scratch[5:]
                   + [pltpu.SemaphoreType.DMA, pltpu.SemaphoreType.DMA])
    res = pl.kernel(
        functools.partial(_segsum_body, with_hist),
        out_type=out_type,
        mesh=_mesh,
        scratch_types=scratch,
    )(x, rows2, cols2, zeros, zrow)
    return res if with_hist else res[0]


def _t2_body(a_ref, rs_ref, w_ref, o_ref):
    s = a_ref[0] + a_ref[1]
    z = jnp.dot(s, w_ref[...], preferred_element_type=_f32) / rs_ref[...]
    o_ref[...] = jnp.where(z > 0, z, jnp.exp(z) - 1.0)


def _t2(acc, rs, w, out_rows, blk):
    return pl.pallas_call(
        _t2_body,
        grid=(out_rows // blk,),
        in_specs=[
            pl.BlockSpec((2, blk, D), lambda i: (0, i, 0)),
            pl.BlockSpec((blk, 1), lambda i: (i, 0)),
            pl.BlockSpec((D, D), lambda i: (0, 0)),
        ],
        out_specs=pl.BlockSpec((blk, D), lambda i: (i, 0)),
        out_shape=jax.ShapeDtypeStruct((out_rows, D), _f32),
    )(acc, rs, w)


def kernel(node_feature, type_feature, length_feature, lane_feature,
           edge_index, struct_adj, struct_assign, fnc_assign, params):
    del struct_adj, struct_assign, fnc_assign

    def pad_idx(a):
        return jnp.pad(a.astype(_i32), (0, NPAD - N))

    def pad_tab(t):
        t = t.astype(_f32)
        return jnp.pad(t, ((0, 0), (0, D - t.shape[1])))

    nf = pad_idx(node_feature)
    tf = pad_idx(type_feature)
    lf = pad_idx(length_feature)
    af = pad_idx(lane_feature)
    pad_i = jnp.arange(EPAD, dtype=_i32)
    rows2 = jnp.concatenate(
        [edge_index[0].astype(_i32), N + pad_i % (ACCR - N)]
    ).reshape(NCHUNK, CH)
    cols2 = jnp.concatenate(
        [edge_index[1].astype(_i32), pad_i % 9973]
    ).reshape(NCHUNK, CH)

    z128 = jnp.zeros((NPAD, D), _f32)
    zrow = jnp.zeros((NPAD,), _f32)

    g = _embed(nf, tf, lf, af,
               pad_tab(params["node_table"]),
               pad_tab(params["type_table"]),
               pad_tab(params["length_table"]),
               pad_tab(params["lane_table"]))
    x1 = _t0(g)

    ws = [p["gat_W"].astype(_f32) for p in params["layers"]]

    acc1, h0, h1 = _segsum(x1, rows2, cols2, z128, zrow, True)
    rs = (h0 + h1).reshape(NPAD, 1)
    x2 = _t2(acc1, rs, ws[0], NPAD, 640)
    acc2 = _segsum(x2, rows2, cols2, z128, zrow, False)
    x3 = _t2(acc2, rs, ws[1], NPAD, 640)
    acc3 = _segsum(x3, rows2, cols2, z128, zrow, False)
    return _t2(acc3, rs, ws[2], N, 400)


# speedup vs baseline: 6.7141x; 1.0153x over previous
import functools

import jax
import jax.numpy as jnp
from jax import lax
from jax.experimental import pallas as pl
from jax.experimental.pallas import tpu as pltpu
from jax.experimental.pallas import tpu_sc as plsc

N = 10000
NPAD = 10240
E = 320000
D = 128
CH = 64
NCHUNK = 5120
EPAD = NCHUNK * CH - E
NC = 2
NT = 16
CPT = NCHUNK // (NC * NT)
PH = CPT // 2
ACCR = 10112
RPT = ACCR // NT

_mesh = plsc.VectorSubcoreMesh(core_axis_name="c", subcore_axis_name="s")

_f32 = jnp.float32
_i32 = jnp.int32


EC = 32
EK = 320 // EC


def _embed_body(nf, tf, lf, af, nt, tt, lt, at_, out, *scr):
    idxs = scr[0:4]
    bufs = [scr[4 + t * 2: 6 + t * 2] for t in range(4)]
    sg = [scr[12 + t * 2: 14 + t * 2] for t in range(4)]
    sw = [scr[20 + t * 2: 22 + t * 2] for t in range(4)]
    tabs = (nt, tt, lt, at_)
    wid = lax.axis_index("s") * NC + lax.axis_index("c")
    wbase = wid * 320
    for t, src in enumerate((nf, tf, lf, af)):
        pltpu.sync_copy(src.at[pl.ds(wid * EK, EK)], idxs[t])

    def gather(t, k, b):
        return pltpu.async_copy(tabs[t].at[idxs[t].at[k, 0]],
                                bufs[t][b], sg[t][b])

    def writeback(t, k, b):
        return pltpu.async_copy(bufs[t][b],
                                out.at[t, pl.ds(wbase + k * EC, EC)],
                                sw[t][b])

    for t in range(4):
        gather(t, 0, 0)
    for k in range(EK):
        b = k % 2
        nb = 1 - b
        for t in range(4):
            pltpu.make_async_copy(tabs[t].at[idxs[t].at[k, 0]],
                                  bufs[t][b], sg[t][b]).wait()
        if k + 1 < EK:
            for t in range(4):
                if k >= 1:
                    pltpu.make_async_copy(
                        bufs[t][nb],
                        out.at[t, pl.ds(wbase + (k - 1) * EC, EC)],
                        sw[t][nb]).wait()
                gather(t, k + 1, nb)
        for t in range(4):
            writeback(t, k, b)
    for t in range(4):
        for k in (EK - 2, EK - 1):
            pltpu.make_async_copy(bufs[t][k % 2],
                                  out.at[t, pl.ds(wbase + k * EC, EC)],
                                  sw[t][k % 2]).wait()


def _embed(nf, tf, lf, af, nt, tt, lt, at_):
    scratch = [pltpu.VMEM((EK, 1, EC), _i32) for _ in range(4)]
    scratch += [pltpu.VMEM((EC, D), _f32) for _ in range(8)]
    scratch += [pltpu.SemaphoreType.DMA for _ in range(16)]
    return pl.kernel(
        _embed_body,
        out_type=jax.ShapeDtypeStruct((4, NPAD, D), _f32),
        mesh=_mesh,
        scratch_types=scratch,
    )(nf, tf, lf, af, nt, tt, lt, at_)


def _t0_body(a_ref, o_ref):
    o_ref[...] = jnp.concatenate(
        [a_ref[0, :, :64], a_ref[1, :, :32],
         a_ref[2, :, :16], a_ref[3, :, :16]], axis=1)


def _t0(g):
    blk = 640
    return pl.pallas_call(
        _t0_body,
        grid=(NPAD // blk,),
        in_specs=[pl.BlockSpec((4, blk, D), lambda i: (0, i, 0))],
        out_specs=pl.BlockSpec((blk, D), lambda i: (i, 0)),
        out_shape=jax.ShapeDtypeStruct((NPAD, D), _f32),
    )(g)


def _segsum_body(with_hist, x, rows2, cols2, zeros, zrow, *args):
    if with_hist:
        (out, out_h0, out_h1, acc, acc_h, rows_all, cols_all,
         vals0, vals1, ones_v, sg0, sg1, ss0, ss1, sh0, sh1) = args
    else:
        (out, acc, rows_all, cols_all,
         vals0, vals1, sg0, sg1, ss0, ss1) = args
    vals = (vals0, vals1)
    sg = (sg0, sg1)
    ss = (ss0, ss1)
    if with_hist:
        sh = (sh0, sh1)
    c = lax.axis_index("c")
    s = lax.axis_index("s")
    rsl = pl.ds(RPT * s, RPT)
    base = (NT * c + s) * CPT
    pltpu.sync_copy(zeros.at[rsl], acc.at[rsl])
    if with_hist:
        for k in range(CH // 16):
            ones_v[pl.ds(16 * k, 16)] = jnp.full((16,), 1.0, _f32)

        @pl.when(s == 0)
        def _():
            pltpu.sync_copy(zrow.at[pl.ds(0, ACCR)], acc_h)

    plsc.subcore_barrier()

    def gather_start(j, b):
        pltpu.async_copy(x.at[cols_all.at[j]], vals[b], sg[b])

    def gather_wait(j, b):
        pltpu.make_async_copy(x.at[cols_all.at[j]], vals[b], sg[b]).wait()

    def scatter_start(j, b):
        pltpu.async_copy(vals[b], acc.at[rows_all.at[j]], ss[b], add=True)
        if with_hist:
            pltpu.async_copy(ones_v, acc_h.at[rows_all.at[j]], sh[b],
                             add=True)

    def scatter_wait(j, b):
        pltpu.make_async_copy(vals[b], acc.at[rows_all.at[j]], ss[b]).wait()
        if with_hist:
            pltpu.make_async_copy(ones_v, acc_h.at[rows_all.at[j]],
                                  sh[b]).wait()

    def step(ji, carry):
        for u in (0, 1):
            j = 2 * ji + u
            nb = 1 - u
            gather_wait(j, u)
            if u == 0:
                @pl.when(j >= 1)
                def _():
                    scatter_wait(j - 1, nb)

                gather_start(j + 1, nb)
            else:
                scatter_wait(j - 1, nb)

                @pl.when(j + 1 < PH)
                def _():
                    gather_start(j + 1, nb)

            scatter_start(j, u)
        return carry

    for p in range(CPT // PH):
        psl = pl.ds(base + p * PH, PH)
        pltpu.sync_copy(rows2.at[psl], rows_all)
        pltpu.sync_copy(cols2.at[psl], cols_all)
        gather_start(0, 0)
        lax.fori_loop(0, PH // 2, step, 0)
        scatter_wait(PH - 1, 1)
    plsc.subcore_barrier()
    pltpu.sync_copy(acc.at[rsl], out.at[c, rsl])
    if with_hist:
        for oh, cc in ((out_h0, 0), (out_h1, 1)):
            @pl.when((c == cc) & (s < 15))
            def _(oh=oh):
                hsl = pl.ds(640 * s, 640)
                pltpu.sync_copy(acc_h.at[hsl], oh.at[hsl])

            @pl.when((c == cc) & (s == 15))
            def _(oh=oh):
                hsl = pl.ds(9600, 512)
                pltpu.sync_copy(acc_h.at[hsl], oh.at[hsl])


def _segsum(x, rows2, cols2, zeros, zrow, with_hist):
    out_type = [jax.ShapeDtypeStruct((2, NPAD, D), _f32)]
    scratch = [
        pltpu.VMEM_SHARED((ACCR, D), _f32),
        pltpu.VMEM((PH, CH), _i32),
        pltpu.VMEM((PH, CH), _i32),
        pltpu.VMEM((CH, D), _f32),
        pltpu.VMEM((CH, D), _f32),
        pltpu.SemaphoreType.DMA,
        pltpu.SemaphoreType.DMA,
        pltpu.SemaphoreType.DMA,
        pltpu.SemaphoreType.DMA,
    ]
    if with_hist:
        out_type += [jax.ShapeDtypeStruct((NPAD,), _f32),
                     jax.ShapeDtypeStruct((NPAD,), _f32)]
        scratch = ([scratch[0], pltpu.VMEM_SHARED((ACCR,), _f32)]
                   + scratch[1:5] + [pltpu.VMEM((CH,), _f32)]
                   + scratch[5:]
                   + [pltpu.SemaphoreType.DMA, pltpu.SemaphoreType.DMA])
    res = pl.kernel(
        functools.partial(_segsum_body, with_hist),
        out_type=out_type,
        mesh=_mesh,
        scratch_types=scratch,
    )(x, rows2, cols2, zeros, zrow)
    return res if with_hist else res[0]


def _t2_body(a_ref, rs_ref, w_ref, o_ref):
    s = a_ref[0] + a_ref[1]
    z = jnp.dot(s, w_ref[...], preferred_element_type=_f32) / rs_ref[...]
    o_ref[...] = jnp.where(z > 0, z, jnp.exp(z) - 1.0)


def _t2(acc, rs, w, out_rows, blk):
    return pl.pallas_call(
        _t2_body,
        grid=(out_rows // blk,),
        in_specs=[
            pl.BlockSpec((2, blk, D), lambda i: (0, i, 0)),
            pl.BlockSpec((blk, 1), lambda i: (i, 0)),
            pl.BlockSpec((D, D), lambda i: (0, 0)),
        ],
        out_specs=pl.BlockSpec((blk, D), lambda i: (i, 0)),
        out_shape=jax.ShapeDtypeStruct((out_rows, D), _f32),
    )(acc, rs, w)


def kernel(node_feature, type_feature, length_feature, lane_feature,
           edge_index, struct_adj, struct_assign, fnc_assign, params):
    del struct_adj, struct_assign, fnc_assign

    def pad_idx(a):
        return jnp.pad(a.astype(_i32), (0, NPAD - N)).reshape(
            NPAD // EC, 1, EC)

    def pad_tab(t):
        t = t.astype(_f32)
        return jnp.pad(t, ((0, 0), (0, D - t.shape[1])))

    nf = pad_idx(node_feature)
    tf = pad_idx(type_feature)
    lf = pad_idx(length_feature)
    af = pad_idx(lane_feature)
    pad_i = jnp.arange(EPAD, dtype=_i32)
    rows2 = jnp.concatenate(
        [edge_index[0].astype(_i32), N + pad_i % (ACCR - N)]
    ).reshape(NCHUNK, CH)
    cols2 = jnp.concatenate(
        [edge_index[1].astype(_i32), pad_i % 9973]
    ).reshape(NCHUNK, CH)

    z128 = jnp.zeros((NPAD, D), _f32)
    zrow = jnp.zeros((NPAD,), _f32)

    g = _embed(nf, tf, lf, af,
               pad_tab(params["node_table"]),
               pad_tab(params["type_table"]),
               pad_tab(params["length_table"]),
               pad_tab(params["lane_table"]))
    x1 = _t0(g)

    ws = [p["gat_W"].astype(_f32) for p in params["layers"]]

    acc1, h0, h1 = _segsum(x1, rows2, cols2, z128, zrow, True)
    rs = (h0 + h1).reshape(NPAD, 1)
    x2 = _t2(acc1, rs, ws[0], NPAD, 640)
    acc2 = _segsum(x2, rows2, cols2, z128, zrow, False)
    x3 = _t2(acc2, rs, ws[1], NPAD, 640)
    acc3 = _segsum(x3, rows2, cols2, z128, zrow, False)
    return _t2(acc3, rs, ws[2], N, 400)


# speedup vs baseline: 7.3858x; 1.1001x over previous
import functools

import jax
import jax.numpy as jnp
from jax import lax
from jax.experimental import pallas as pl
from jax.experimental.pallas import tpu as pltpu
from jax.experimental.pallas import tpu_sc as plsc

N = 10000
NPAD = 10240
E = 320000
D = 128
CH = 64
NCHUNK = 5120
EPAD = NCHUNK * CH - E
NC = 2
NT = 16
CPT = NCHUNK // (NC * NT)
PH = CPT // 2
ACCR = 10112
RPT = ACCR // NT

_mesh = plsc.VectorSubcoreMesh(core_axis_name="c", subcore_axis_name="s")

_f32 = jnp.float32
_i32 = jnp.int32


EC = 64
EK = 320 // EC


def _embed_body(nf, nt, out, idx, buf0, buf1, sg0, sg1, sw0, sw1):
    bufs = (buf0, buf1)
    sg = (sg0, sg1)
    sw = (sw0, sw1)
    wid = lax.axis_index("s") * NC + lax.axis_index("c")
    wbase = wid * 320
    pltpu.sync_copy(nf.at[pl.ds(wid * EK, EK)], idx)

    def gather(k, b):
        pltpu.async_copy(nt.at[idx.at[k, 0]], bufs[b], sg[b])

    def wb(k, b):
        return pltpu.make_async_copy(
            bufs[b], out.at[pl.ds(wbase + k * EC, EC)], sw[b])

    gather(0, 0)
    for k in range(EK):
        b = k % 2
        nb = 1 - b
        pltpu.make_async_copy(nt.at[idx.at[k, 0]], bufs[b], sg[b]).wait()
        if k + 1 < EK:
            if k >= 1:
                wb(k - 1, nb).wait()
            gather(k + 1, nb)
        pltpu.async_copy(bufs[b], out.at[pl.ds(wbase + k * EC, EC)], sw[b])
    for k in (EK - 2, EK - 1):
        wb(k, k % 2).wait()


def _embed(nf, nt):
    return pl.kernel(
        _embed_body,
        out_type=jax.ShapeDtypeStruct((NPAD, D), _f32),
        mesh=_mesh,
        scratch_types=[
            pltpu.VMEM((EK, 1, EC), _i32),
            pltpu.VMEM((EC, D), _f32),
            pltpu.VMEM((EC, D), _f32),
            pltpu.SemaphoreType.DMA,
            pltpu.SemaphoreType.DMA,
            pltpu.SemaphoreType.DMA,
            pltpu.SemaphoreType.DMA,
        ],
    )(nf, nt)


def _onehot_lookup(idx_col, tab_ref, ncls):
    npd = tab_ref.shape[0]
    oh = (idx_col == lax.broadcasted_iota(_i32, (idx_col.shape[0], npd), 1))
    del ncls
    return jnp.dot(oh.astype(_f32), tab_ref[...],
                   preferred_element_type=_f32)


def _t0_body(g_ref, tf_ref, lf_ref, af_ref, tt_ref, lt_ref, at_ref, o_ref):
    o_ref[...] = jnp.concatenate(
        [g_ref[:, :64],
         _onehot_lookup(tf_ref[...], tt_ref, 20),
         _onehot_lookup(lf_ref[...], lt_ref, 100),
         _onehot_lookup(af_ref[...], at_ref, 10)], axis=1)


def _t0(g, tf2, lf2, af2, tt_p, lt_p, at_p):
    blk = 640
    return pl.pallas_call(
        _t0_body,
        grid=(NPAD // blk,),
        in_specs=[
            pl.BlockSpec((blk, D), lambda i: (i, 0)),
            pl.BlockSpec((blk, 1), lambda i: (i, 0)),
            pl.BlockSpec((blk, 1), lambda i: (i, 0)),
            pl.BlockSpec((blk, 1), lambda i: (i, 0)),
            pl.BlockSpec((32, 32), lambda i: (0, 0)),
            pl.BlockSpec((128, 16), lambda i: (0, 0)),
            pl.BlockSpec((16, 16), lambda i: (0, 0)),
        ],
        out_specs=pl.BlockSpec((blk, D), lambda i: (i, 0)),
        out_shape=jax.ShapeDtypeStruct((NPAD, D), _f32),
    )(g, tf2, lf2, af2, tt_p, lt_p, at_p)


def _segsum_body(with_hist, x, rows2, cols2, zeros, zrow, *args):
    if with_hist:
        (out, out_h0, out_h1, acc, acc_h, rows_all, cols_all,
         vals0, vals1, ones_v, sg0, sg1, ss0, ss1, sh0, sh1) = args
    else:
        (out, acc, rows_all, cols_all,
         vals0, vals1, sg0, sg1, ss0, ss1) = args
    vals = (vals0, vals1)
    sg = (sg0, sg1)
    ss = (ss0, ss1)
    if with_hist:
        sh = (sh0, sh1)
    c = lax.axis_index("c")
    s = lax.axis_index("s")
    rsl = pl.ds(RPT * s, RPT)
    base = (NT * c + s) * CPT
    pltpu.sync_copy(zeros.at[rsl], acc.at[rsl])
    if with_hist:
        for k in range(CH // 16):
            ones_v[pl.ds(16 * k, 16)] = jnp.full((16,), 1.0, _f32)

        @pl.when(s == 0)
        def _():
            pltpu.sync_copy(zrow.at[pl.ds(0, ACCR)], acc_h)

    plsc.subcore_barrier()

    def gather_start(j, b):
        pltpu.async_copy(x.at[cols_all.at[j]], vals[b], sg[b])

    def gather_wait(j, b):
        pltpu.make_async_copy(x.at[cols_all.at[j]], vals[b], sg[b]).wait()

    def scatter_start(j, b):
        pltpu.async_copy(vals[b], acc.at[rows_all.at[j]], ss[b], add=True)
        if with_hist:
            pltpu.async_copy(ones_v, acc_h.at[rows_all.at[j]], sh[b],
                             add=True)

    def scatter_wait(j, b):
        pltpu.make_async_copy(vals[b], acc.at[rows_all.at[j]], ss[b]).wait()
        if with_hist:
            pltpu.make_async_copy(ones_v, acc_h.at[rows_all.at[j]],
                                  sh[b]).wait()

    def step(ji, carry):
        for u in (0, 1):
            j = 2 * ji + u
            nb = 1 - u
            gather_wait(j, u)
            if u == 0:
                @pl.when(j >= 1)
                def _():
                    scatter_wait(j - 1, nb)

                gather_start(j + 1, nb)
            else:
                scatter_wait(j - 1, nb)

                @pl.when(j + 1 < PH)
                def _():
                    gather_start(j + 1, nb)

            scatter_start(j, u)
        return carry

    for p in range(CPT // PH):
        psl = pl.ds(base + p * PH, PH)
        pltpu.sync_copy(rows2.at[psl], rows_all)
        pltpu.sync_copy(cols2.at[psl], cols_all)
        gather_start(0, 0)
        lax.fori_loop(0, PH // 2, step, 0)
        scatter_wait(PH - 1, 1)
    plsc.subcore_barrier()
    pltpu.sync_copy(acc.at[rsl], out.at[c, rsl])
    if with_hist:
        for oh, cc in ((out_h0, 0), (out_h1, 1)):
            @pl.when((c == cc) & (s < 15))
            def _(oh=oh):
                hsl = pl.ds(640 * s, 640)
                pltpu.sync_copy(acc_h.at[hsl], oh.at[hsl])

            @pl.when((c == cc) & (s == 15))
            def _(oh=oh):
                hsl = pl.ds(9600, 512)
                pltpu.sync_copy(acc_h.at[hsl], oh.at[hsl])


def _segsum(x, rows2, cols2, zeros, zrow, with_hist):
    out_type = [jax.ShapeDtypeStruct((2, NPAD, D), _f32)]
    scratch = [
        pltpu.VMEM_SHARED((ACCR, D), _f32),
        pltpu.VMEM((PH, CH), _i32),
        pltpu.VMEM((PH, CH), _i32),
        pltpu.VMEM((CH, D), _f32),
        pltpu.VMEM((CH, D), _f32),
        pltpu.SemaphoreType.DMA,
        pltpu.SemaphoreType.DMA,
        pltpu.SemaphoreType.DMA,
        pltpu.SemaphoreType.DMA,
    ]
    if with_hist:
        out_type += [jax.ShapeDtypeStruct((NPAD,), _f32),
                     jax.ShapeDtypeStruct((NPAD,), _f32)]
        scratch = ([scratch[0], pltpu.VMEM_SHARED((ACCR,), _f32)]
                   + scratch[1:5] + [pltpu.VMEM((CH,), _f32)]
                   + scratch[5:]
                   + [pltpu.SemaphoreType.DMA, pltpu.SemaphoreType.DMA])
    res = pl.kernel(
        functools.partial(_segsum_body, with_hist),
        out_type=out_type,
        mesh=_mesh,
        scratch_types=scratch,
    )(x, rows2, cols2, zeros, zrow)
    return res if with_hist else res[0]


def _t2_body(a_ref, rs_ref, w_ref, o_ref):
    s = a_ref[0] + a_ref[1]
    z = jnp.dot(s, w_ref[...], preferred_element_type=_f32) / rs_ref[...]
    o_ref[...] = jnp.where(z > 0, z, jnp.exp(z) - 1.0)


def _t2(acc, rs, w, out_rows, blk):
    return pl.pallas_call(
        _t2_body,
        grid=(out_rows // blk,),
        in_specs=[
            pl.BlockSpec((2, blk, D), lambda i: (0, i, 0)),
            pl.BlockSpec((blk, 1), lambda i: (i, 0)),
            pl.BlockSpec((D, D), lambda i: (0, 0)),
        ],
        out_specs=pl.BlockSpec((blk, D), lambda i: (i, 0)),
        out_shape=jax.ShapeDtypeStruct((out_rows, D), _f32),
    )(acc, rs, w)


def kernel(node_feature, type_feature, length_feature, lane_feature,
           edge_index, struct_adj, struct_assign, fnc_assign, params):
    del struct_adj, struct_assign, fnc_assign

    nf = jnp.pad(node_feature.astype(_i32), (0, NPAD - N)).reshape(
        NPAD // EC, 1, EC)
    tf2 = jnp.pad(type_feature.astype(_i32), (0, NPAD - N)).reshape(NPAD, 1)
    lf2 = jnp.pad(length_feature.astype(_i32), (0, NPAD - N)).reshape(NPAD, 1)
    af2 = jnp.pad(lane_feature.astype(_i32), (0, NPAD - N)).reshape(NPAD, 1)
    pad_i = jnp.arange(EPAD, dtype=_i32)
    rows2 = jnp.concatenate(
        [edge_index[0].astype(_i32), N + pad_i % (ACCR - N)]
    ).reshape(NCHUNK, CH)
    cols2 = jnp.concatenate(
        [edge_index[1].astype(_i32), pad_i % 9973]
    ).reshape(NCHUNK, CH)

    z128 = jnp.zeros((NPAD, D), _f32)
    zrow = jnp.zeros((NPAD,), _f32)

    nt_p = jnp.pad(params["node_table"].astype(_f32), ((0, 0), (0, 64)))
    tt_p = jnp.pad(params["type_table"].astype(_f32), ((0, 12), (0, 0)))
    lt_p = jnp.pad(params["length_table"].astype(_f32), ((0, 28), (0, 0)))
    at_p = jnp.pad(params["lane_table"].astype(_f32), ((0, 6), (0, 0)))

    g = _embed(nf, nt_p)
    x1 = _t0(g, tf2, lf2, af2, tt_p, lt_p, at_p)

    ws = [p["gat_W"].astype(_f32) for p in params["layers"]]

    acc1, h0, h1 = _segsum(x1, rows2, cols2, z128, zrow, True)
    rs = (h0 + h1).reshape(NPAD, 1)
    x2 = _t2(acc1, rs, ws[0], NPAD, 640)
    acc2 = _segsum(x2, rows2, cols2, z128, zrow, False)
    x3 = _t2(acc2, rs, ws[1], NPAD, 640)
    acc3 = _segsum(x3, rows2, cols2, z128, zrow, False)
    return _t2(acc3, rs, ws[2], N, 400)


# speedup vs baseline: 7.6816x; 1.0400x over previous
import functools

import jax
import jax.numpy as jnp
from jax import lax
from jax.experimental import pallas as pl
from jax.experimental.pallas import tpu as pltpu
from jax.experimental.pallas import tpu_sc as plsc

N = 10000
NPAD = 10240
E = 320000
D = 128
CH = 64
NCHUNK = 5120
EPAD = NCHUNK * CH - E
NC = 2
NT = 16
CPT = NCHUNK // (NC * NT)
PH = CPT // 2
ACCR = 10112
RPT = ACCR // NT

_mesh = plsc.VectorSubcoreMesh(core_axis_name="c", subcore_axis_name="s")

_f32 = jnp.float32
_i32 = jnp.int32


EC = 64
EK = 320 // EC


def _embed_body(nf, nt, out, idx, buf0, buf1, sg0, sg1, sw0, sw1):
    bufs = (buf0, buf1)
    sg = (sg0, sg1)
    sw = (sw0, sw1)
    wid = lax.axis_index("s") * NC + lax.axis_index("c")
    wbase = wid * 320
    pltpu.sync_copy(nf.at[pl.ds(wid * EK, EK)], idx)

    def gather(k, b):
        pltpu.async_copy(nt.at[idx.at[k, 0]], bufs[b], sg[b])

    def wb(k, b):
        return pltpu.make_async_copy(
            bufs[b], out.at[pl.ds(wbase + k * EC, EC)], sw[b])

    gather(0, 0)
    for k in range(EK):
        b = k % 2
        nb = 1 - b
        pltpu.make_async_copy(nt.at[idx.at[k, 0]], bufs[b], sg[b]).wait()
        if k + 1 < EK:
            if k >= 1:
                wb(k - 1, nb).wait()
            gather(k + 1, nb)
        pltpu.async_copy(bufs[b], out.at[pl.ds(wbase + k * EC, EC)], sw[b])
    for k in (EK - 2, EK - 1):
        wb(k, k % 2).wait()


def _embed(nf, nt):
    return pl.kernel(
        _embed_body,
        out_type=jax.ShapeDtypeStruct((NPAD, D), _f32),
        mesh=_mesh,
        scratch_types=[
            pltpu.VMEM((EK, 1, EC), _i32),
            pltpu.VMEM((EC, D), _f32),
            pltpu.VMEM((EC, D), _f32),
            pltpu.SemaphoreType.DMA,
            pltpu.SemaphoreType.DMA,
            pltpu.SemaphoreType.DMA,
            pltpu.SemaphoreType.DMA,
        ],
    )(nf, nt)


def _onehot_lookup(idx_col, tab_ref, ncls):
    npd = tab_ref.shape[0]
    oh = (idx_col == lax.broadcasted_iota(_i32, (idx_col.shape[0], npd), 1))
    del ncls
    return jnp.dot(oh.astype(_f32), tab_ref[...],
                   preferred_element_type=_f32)


def _t0_body(g_ref, tf_ref, lf_ref, af_ref, tt_ref, lt_ref, at_ref, o_ref):
    o_ref[...] = jnp.concatenate(
        [g_ref[:, :64],
         _onehot_lookup(tf_ref[...], tt_ref, 20),
         _onehot_lookup(lf_ref[...], lt_ref, 100),
         _onehot_lookup(af_ref[...], at_ref, 10)], axis=1)


def _t0(g, tf2, lf2, af2, tt_p, lt_p, at_p):
    blk = 2560
    return pl.pallas_call(
        _t0_body,
        grid=(NPAD // blk,),
        in_specs=[
            pl.BlockSpec((blk, D), lambda i: (i, 0)),
            pl.BlockSpec((blk, 1), lambda i: (i, 0)),
            pl.BlockSpec((blk, 1), lambda i: (i, 0)),
            pl.BlockSpec((blk, 1), lambda i: (i, 0)),
            pl.BlockSpec((32, 32), lambda i: (0, 0)),
            pl.BlockSpec((128, 16), lambda i: (0, 0)),
            pl.BlockSpec((16, 16), lambda i: (0, 0)),
        ],
        out_specs=pl.BlockSpec((blk, D), lambda i: (i, 0)),
        out_shape=jax.ShapeDtypeStruct((NPAD, D), _f32),
    )(g, tf2, lf2, af2, tt_p, lt_p, at_p)


def _segsum_body(with_hist, x, rows2, cols2, zeros, zrow, *args):
    if with_hist:
        (out, out_h0, out_h1, acc, acc_h, rows_all, cols_all,
         vals0, vals1, ones_v, sg0, sg1, ss0, ss1, sh0, sh1) = args
    else:
        (out, acc, rows_all, cols_all,
         vals0, vals1, sg0, sg1, ss0, ss1) = args
    vals = (vals0, vals1)
    sg = (sg0, sg1)
    ss = (ss0, ss1)
    if with_hist:
        sh = (sh0, sh1)
    c = lax.axis_index("c")
    s = lax.axis_index("s")
    rsl = pl.ds(RPT * s, RPT)
    base = (NT * c + s) * CPT
    pltpu.sync_copy(zeros.at[rsl], acc.at[rsl])
    if with_hist:
        for k in range(CH // 16):
            ones_v[pl.ds(16 * k, 16)] = jnp.full((16,), 1.0, _f32)

        @pl.when(s == 0)
        def _():
            pltpu.sync_copy(zrow.at[pl.ds(0, ACCR)], acc_h)

    plsc.subcore_barrier()

    def gather_start(j, b):
        pltpu.async_copy(x.at[cols_all.at[j]], vals[b], sg[b])

    def gather_wait(j, b):
        pltpu.make_async_copy(x.at[cols_all.at[j]], vals[b], sg[b]).wait()

    def scatter_start(j, b):
        pltpu.async_copy(vals[b], acc.at[rows_all.at[j]], ss[b], add=True)
        if with_hist:
            pltpu.async_copy(ones_v, acc_h.at[rows_all.at[j]], sh[b],
                             add=True)

    def scatter_wait(j, b):
        pltpu.make_async_copy(vals[b], acc.at[rows_all.at[j]], ss[b]).wait()
        if with_hist:
            pltpu.make_async_copy(ones_v, acc_h.at[rows_all.at[j]],
                                  sh[b]).wait()

    def step(ji, carry):
        for u in (0, 1):
            j = 2 * ji + u
            nb = 1 - u
            gather_wait(j, u)
            if u == 0:
                @pl.when(j >= 1)
                def _():
                    scatter_wait(j - 1, nb)

                gather_start(j + 1, nb)
            else:
                scatter_wait(j - 1, nb)

                @pl.when(j + 1 < PH)
                def _():
                    gather_start(j + 1, nb)

            scatter_start(j, u)
        return carry

    for p in range(CPT // PH):
        psl = pl.ds(base + p * PH, PH)
        pltpu.sync_copy(rows2.at[psl], rows_all)
        pltpu.sync_copy(cols2.at[psl], cols_all)
        gather_start(0, 0)
        lax.fori_loop(0, PH // 2, step, 0)
        scatter_wait(PH - 1, 1)
    plsc.subcore_barrier()
    pltpu.sync_copy(acc.at[rsl], out.at[c, rsl])
    if with_hist:
        for oh, cc in ((out_h0, 0), (out_h1, 1)):
            @pl.when((c == cc) & (s < 15))
            def _(oh=oh):
                hsl = pl.ds(640 * s, 640)
                pltpu.sync_copy(acc_h.at[hsl], oh.at[hsl])

            @pl.when((c == cc) & (s == 15))
            def _(oh=oh):
                hsl = pl.ds(9600, 512)
                pltpu.sync_copy(acc_h.at[hsl], oh.at[hsl])


def _segsum(x, rows2, cols2, zeros, zrow, with_hist):
    out_type = [jax.ShapeDtypeStruct((2, NPAD, D), _f32)]
    scratch = [
        pltpu.VMEM_SHARED((ACCR, D), _f32),
        pltpu.VMEM((PH, CH), _i32),
        pltpu.VMEM((PH, CH), _i32),
        pltpu.VMEM((CH, D), _f32),
        pltpu.VMEM((CH, D), _f32),
        pltpu.SemaphoreType.DMA,
        pltpu.SemaphoreType.DMA,
        pltpu.SemaphoreType.DMA,
        pltpu.SemaphoreType.DMA,
    ]
    if with_hist:
        out_type += [jax.ShapeDtypeStruct((NPAD,), _f32),
                     jax.ShapeDtypeStruct((NPAD,), _f32)]
        scratch = ([scratch[0], pltpu.VMEM_SHARED((ACCR,), _f32)]
                   + scratch[1:5] + [pltpu.VMEM((CH,), _f32)]
                   + scratch[5:]
                   + [pltpu.SemaphoreType.DMA, pltpu.SemaphoreType.DMA])
    res = pl.kernel(
        functools.partial(_segsum_body, with_hist),
        out_type=out_type,
        mesh=_mesh,
        scratch_types=scratch,
    )(x, rows2, cols2, zeros, zrow)
    return res if with_hist else res[0]


def _t2_body(a_ref, rs_ref, w_ref, o_ref):
    s = a_ref[0] + a_ref[1]
    z = jnp.dot(s, w_ref[...], preferred_element_type=_f32) / rs_ref[...]
    o_ref[...] = jnp.where(z > 0, z, jnp.exp(z) - 1.0)


def _t2(acc, rs, w, out_rows, blk):
    return pl.pallas_call(
        _t2_body,
        grid=(out_rows // blk,),
        in_specs=[
            pl.BlockSpec((2, blk, D), lambda i: (0, i, 0)),
            pl.BlockSpec((blk, 1), lambda i: (i, 0)),
            pl.BlockSpec((D, D), lambda i: (0, 0)),
        ],
        out_specs=pl.BlockSpec((blk, D), lambda i: (i, 0)),
        out_shape=jax.ShapeDtypeStruct((out_rows, D), _f32),
    )(acc, rs, w)


def kernel(node_feature, type_feature, length_feature, lane_feature,
           edge_index, struct_adj, struct_assign, fnc_assign, params):
    del struct_adj, struct_assign, fnc_assign

    nf = jnp.pad(node_feature.astype(_i32), (0, NPAD - N)).reshape(
        NPAD // EC, 1, EC)
    tf2 = jnp.pad(type_feature.astype(_i32), (0, NPAD - N)).reshape(NPAD, 1)
    lf2 = jnp.pad(length_feature.astype(_i32), (0, NPAD - N)).reshape(NPAD, 1)
    af2 = jnp.pad(lane_feature.astype(_i32), (0, NPAD - N)).reshape(NPAD, 1)
    pad_i = jnp.arange(EPAD, dtype=_i32)
    rows2 = jnp.concatenate(
        [edge_index[0].astype(_i32), N + pad_i % (ACCR - N)]
    ).reshape(NCHUNK, CH)
    cols2 = jnp.concatenate(
        [edge_index[1].astype(_i32), pad_i % 9973]
    ).reshape(NCHUNK, CH)

    z128 = jnp.zeros((NPAD, D), _f32)
    zrow = jnp.zeros((NPAD,), _f32)

    nt_p = jnp.pad(params["node_table"].astype(_f32), ((0, 0), (0, 64)))
    tt_p = jnp.pad(params["type_table"].astype(_f32), ((0, 12), (0, 0)))
    lt_p = jnp.pad(params["length_table"].astype(_f32), ((0, 28), (0, 0)))
    at_p = jnp.pad(params["lane_table"].astype(_f32), ((0, 6), (0, 0)))

    g = _embed(nf, nt_p)
    x1 = _t0(g, tf2, lf2, af2, tt_p, lt_p, at_p)

    ws = [p["gat_W"].astype(_f32) for p in params["layers"]]

    acc1, h0, h1 = _segsum(x1, rows2, cols2, z128, zrow, True)
    rs = (h0 + h1).reshape(NPAD, 1)
    x2 = _t2(acc1, rs, ws[0], NPAD, 2560)
    acc2 = _segsum(x2, rows2, cols2, z128, zrow, False)
    x3 = _t2(acc2, rs, ws[1], NPAD, 2560)
    acc3 = _segsum(x3, rows2, cols2, z128, zrow, False)
    return _t2(acc3, rs, ws[2], N, 2000)
